# baseline (speedup 1.0000x reference)
import functools

import jax
import jax.numpy as jnp
from jax import lax
from jax.experimental import pallas as pl
from jax.experimental.pallas import tpu as pltpu
from jax.experimental.pallas import tpu_sc as plsc

VOCAB = 100000
HIST_LEN = 200
BATCH = 16384
PADDING_IDX = 0

NUM_CORES = 2
NUM_SUBCORES = 16
LANES = 16
NW = NUM_CORES * NUM_SUBCORES
ROWS_PER_W = BATCH // NW
CHUNK = 128
NCHUNK = ROWS_PER_W // CHUNK

_WIN_OFFS = tuple(16 * j for j in range(12)) + (HIST_LEN - LANES,)

@functools.cache
def _build_history_view():
    mesh = plsc.VectorSubcoreMesh(core_axis_name="c", subcore_axis_name="s")

    @functools.partial(
        pl.kernel,
        out_type=(
            jax.ShapeDtypeStruct((BATCH, HIST_LEN), jnp.int32),
            jax.ShapeDtypeStruct((BATCH, HIST_LEN), jnp.int32),
        ),
        mesh=mesh,
        compiler_params=pltpu.CompilerParams(
            use_tc_tiling_on_sc=False,
            needs_layout_passes=False,
        ),
        scratch_types=[
            pltpu.VMEM((CHUNK,), jnp.int32),
            pltpu.VMEM((CHUNK, LANES), jnp.int32),
            pltpu.VMEM((CHUNK, HIST_LEN), jnp.int32),
            pltpu.VMEM((CHUNK, HIST_LEN), jnp.int32),
            pltpu.SemaphoreType.DMA,
        ],
    )
    def _history_view(hist_hbm, anchor_hbm, tgtb_hbm, padded_hbm, mask_hbm,
                      idx_v, tgt_v, rows_v, mask_v, sem):
        wid = lax.axis_index("s") * NUM_CORES + lax.axis_index("c")

        def row_body(r, _):
            tgt = tgt_v[r, pl.ds(0, LANES)]
            for off in _WIN_OFFS:
                h = rows_v[r, pl.ds(off, LANES)]
                keep = ~((h == tgt) | (h == PADDING_IDX))
                rows_v[r, pl.ds(off, LANES)] = jnp.where(keep, h, PADDING_IDX)
                mask_v[r, pl.ds(off, LANES)] = keep.astype(jnp.int32)
            return 0

        for c in range(NCHUNK):
            base = wid * ROWS_PER_W + c * CHUNK
            pltpu.sync_copy(anchor_hbm.at[pl.ds(base, CHUNK)], idx_v)
            pltpu.sync_copy(tgtb_hbm.at[pl.ds(base, CHUNK)], tgt_v)
            pltpu.async_copy(hist_hbm.at[idx_v], rows_v, sem).wait()
            lax.fori_loop(0, CHUNK, row_body, 0)
            pltpu.sync_copy(rows_v, padded_hbm.at[pl.ds(base, CHUNK)])
            pltpu.sync_copy(mask_v, mask_hbm.at[pl.ds(base, CHUNK)])

    return _history_view


def kernel(histories, anchor_idx, target_idx):
    out_dtype = histories.dtype
    tgt_bcast = jnp.broadcast_to(
        target_idx.astype(jnp.int32)[:, None], (BATCH, LANES))
    zero = jax.lax.optimization_barrier(jnp.int32(0))
    hist_lin = jnp.bitwise_xor(histories.astype(jnp.int32), zero)
    padded, mask_i32 = _build_history_view()(
        hist_lin,
        anchor_idx.astype(jnp.int32),
        tgt_bcast,
    )
    return padded.astype(out_dtype), mask_i32.astype(jnp.bool_)


# speedup vs baseline: 1.3402x; 1.3402x over previous
import functools

import jax
import jax.numpy as jnp
from jax import lax
from jax.experimental import pallas as pl
from jax.experimental.pallas import tpu as pltpu
from jax.experimental.pallas import tpu_sc as plsc

VOCAB = 100000
HIST_LEN = 200
PAD_LEN = 256
BATCH = 16384
PADDING_IDX = 0

NUM_CORES = 2
NUM_SUBCORES = 16
LANES = 16
NW = NUM_CORES * NUM_SUBCORES
ROWS_PER_W = BATCH // NW
CHUNK = 128
NCHUNK = ROWS_PER_W // CHUNK

_WIN_OFFS = tuple(16 * j for j in range(12)) + (HIST_LEN - LANES,)


@functools.cache
def _build_history_view():
    mesh = plsc.VectorSubcoreMesh(core_axis_name="c", subcore_axis_name="s")

    @functools.partial(
        pl.kernel,
        out_type=(
            jax.ShapeDtypeStruct((BATCH, PAD_LEN), jnp.int32),
            jax.ShapeDtypeStruct((BATCH, PAD_LEN), jnp.int32),
        ),
        mesh=mesh,
        compiler_params=pltpu.CompilerParams(
            use_tc_tiling_on_sc=True,
            needs_layout_passes=False,
        ),
        scratch_types=[
            pltpu.VMEM((CHUNK,), jnp.int32),
            pltpu.VMEM((CHUNK * LANES,), jnp.int32),
            pltpu.VMEM((CHUNK, PAD_LEN), jnp.int32),
            pltpu.VMEM((CHUNK, PAD_LEN), jnp.int32),
            pltpu.SemaphoreType.DMA,
        ],
    )
    def _history_view(hist_hbm, anchor_hbm, tgtb_hbm, padded_hbm, mask_hbm,
                      idx_v, tgt_v, rows_v, mask_v, sem):
        wid = lax.axis_index("s") * NUM_CORES + lax.axis_index("c")

        def row_body(r, _):
            tgt = tgt_v[pl.ds(r * LANES, LANES)]
            for off in _WIN_OFFS:
                h = rows_v[r, pl.ds(off, LANES)]
                keep = ~((h == tgt) | (h == PADDING_IDX))
                rows_v[r, pl.ds(off, LANES)] = jnp.where(keep, h, PADDING_IDX)
                mask_v[r, pl.ds(off, LANES)] = keep.astype(jnp.int32)
            return 0

        for c in range(NCHUNK):
            base = wid * ROWS_PER_W + c * CHUNK
            pltpu.sync_copy(anchor_hbm.at[pl.ds(base, CHUNK)], idx_v)
            pltpu.sync_copy(tgtb_hbm.at[pl.ds(base * LANES, CHUNK * LANES)],
                            tgt_v)
            pltpu.async_copy(hist_hbm.at[idx_v], rows_v, sem).wait()
            lax.fori_loop(0, CHUNK, row_body, 0)
            pltpu.sync_copy(rows_v, padded_hbm.at[pl.ds(base, CHUNK)])
            pltpu.sync_copy(mask_v, mask_hbm.at[pl.ds(base, CHUNK)])

    return _history_view


def kernel(histories, anchor_idx, target_idx):
    out_dtype = histories.dtype
    hist_pad = jnp.pad(histories.astype(jnp.int32),
                       ((0, 0), (0, PAD_LEN - HIST_LEN)))
    tgt_flat = jnp.broadcast_to(
        target_idx.astype(jnp.int32)[:, None],
        (BATCH, LANES)).reshape(BATCH * LANES)
    padded, mask_i32 = _build_history_view()(
        hist_pad,
        anchor_idx.astype(jnp.int32),
        tgt_flat,
    )
    return (padded[:, :HIST_LEN].astype(out_dtype),
            mask_i32[:, :HIST_LEN].astype(jnp.bool_))


# speedup vs baseline: 1.3418x; 1.0012x over previous
import functools

import jax
import jax.numpy as jnp
from jax import lax
from jax.experimental import pallas as pl
from jax.experimental.pallas import tpu as pltpu
from jax.experimental.pallas import tpu_sc as plsc

VOCAB = 100000
HIST_LEN = 200
PAD_LEN = 256
BATCH = 16384
PADDING_IDX = 0

NUM_CORES = 2
NUM_SUBCORES = 16
LANES = 16
NW = NUM_CORES * NUM_SUBCORES
ROWS_PER_W = BATCH // NW
CHUNK = 128
NCHUNK = ROWS_PER_W // CHUNK

_WIN_OFFS = tuple(16 * j for j in range(12)) + (HIST_LEN - LANES,)


@functools.cache
def _build_history_view():
    mesh = plsc.VectorSubcoreMesh(core_axis_name="c", subcore_axis_name="s")

    @functools.partial(
        pl.kernel,
        out_type=(
            jax.ShapeDtypeStruct((BATCH, PAD_LEN), jnp.int32),
            jax.ShapeDtypeStruct((BATCH, PAD_LEN), jnp.int32),
        ),
        mesh=mesh,
        compiler_params=pltpu.CompilerParams(
            use_tc_tiling_on_sc=True,
            needs_layout_passes=False,
        ),
        scratch_types=[
            pltpu.VMEM((CHUNK,), jnp.int32),
            pltpu.VMEM((CHUNK * LANES,), jnp.int32),
            pltpu.VMEM((CHUNK, PAD_LEN), jnp.int32),
            pltpu.VMEM((CHUNK, PAD_LEN), jnp.int32),
            pltpu.SemaphoreType.DMA,
        ],
    )
    def _history_view(hist_hbm, anchor_hbm, tgtb_hbm, padded_hbm, mask_hbm,
                      idx_v, tgt_v, rows_v, mask_v, sem):
        wid = lax.axis_index("s") * NUM_CORES + lax.axis_index("c")

        def row_body(r, _):
            tgt = tgt_v[pl.ds(r * LANES, LANES)]
            for off in _WIN_OFFS:
                h = rows_v[r, pl.ds(off, LANES)]
                keep = ~((h == tgt) | (h == PADDING_IDX))
                rows_v[r, pl.ds(off, LANES)] = jnp.where(keep, h, PADDING_IDX)
                mask_v[r, pl.ds(off, LANES)] = keep.astype(jnp.int32)
            return 0

        for c in range(NCHUNK):
            base = wid * ROWS_PER_W + c * CHUNK
            pltpu.sync_copy(anchor_hbm.at[pl.ds(base, CHUNK)], idx_v)
            pltpu.sync_copy(tgtb_hbm.at[pl.ds(base * LANES, CHUNK * LANES)],
                            tgt_v)
            pltpu.async_copy(hist_hbm.at[idx_v], rows_v, sem).wait()
            lax.fori_loop(0, CHUNK, row_body, 0)
            pltpu.sync_copy(rows_v, padded_hbm.at[pl.ds(base, CHUNK)])
            pltpu.sync_copy(mask_v, mask_hbm.at[pl.ds(base, CHUNK)])

    return _history_view


def kernel(histories, anchor_idx, target_idx):
    out_dtype = histories.dtype
    zero = lax.optimization_barrier(jnp.int32(0))
    hist_pad = jnp.pad(histories.astype(jnp.int32),
                       ((0, 0), (0, PAD_LEN - HIST_LEN))) ^ zero
    tgt_flat = jnp.broadcast_to(
        target_idx.astype(jnp.int32)[:, None],
        (BATCH, LANES)).reshape(BATCH * LANES)
    padded, mask_i32 = _build_history_view()(
        hist_pad,
        anchor_idx.astype(jnp.int32),
        tgt_flat,
    )
    return (padded[:, :HIST_LEN].astype(out_dtype),
            mask_i32[:, :HIST_LEN].astype(jnp.bool_))


# speedup vs baseline: 4.5351x; 3.3799x over previous
import functools

import jax
import jax.numpy as jnp
from jax import lax
from jax.experimental import pallas as pl
from jax.experimental.pallas import tpu as pltpu
from jax.experimental.pallas import tpu_sc as plsc

VOCAB = 100000
HIST_LEN = 200
PAD_LEN = 256
BATCH = 16384
PADDING_IDX = 0

NUM_CORES = 2
NUM_SUBCORES = 16
LANES = 16
NW = NUM_CORES * NUM_SUBCORES
ROWS_PER_W = BATCH // NW
CHUNK = 128
NCHUNK = ROWS_PER_W // CHUNK

_WIN_OFFS = tuple(16 * j for j in range(12)) + (HIST_LEN - LANES,)


TBLK = 2048


def _transpose_body(in_ref, out_ref):
    x = in_ref[...]
    out_ref[:, :HIST_LEN] = x.T


@functools.cache
def _build_transpose():
    grid = (VOCAB + TBLK - 1) // TBLK
    return pl.pallas_call(
        _transpose_body,
        grid=(grid,),
        in_specs=[pl.BlockSpec((HIST_LEN, TBLK), lambda i: (0, i))],
        out_specs=pl.BlockSpec((TBLK, PAD_LEN), lambda i: (i, 0)),
        out_shape=jax.ShapeDtypeStruct((VOCAB, PAD_LEN), jnp.int32),
    )


@functools.cache
def _build_history_view():
    mesh = plsc.VectorSubcoreMesh(core_axis_name="c", subcore_axis_name="s")

    @functools.partial(
        pl.kernel,
        out_type=(
            jax.ShapeDtypeStruct((BATCH, PAD_LEN), jnp.int32),
            jax.ShapeDtypeStruct((BATCH, PAD_LEN), jnp.int32),
        ),
        mesh=mesh,
        compiler_params=pltpu.CompilerParams(
            use_tc_tiling_on_sc=True,
            needs_layout_passes=False,
        ),
        scratch_types=[
            pltpu.VMEM((CHUNK,), jnp.int32),
            pltpu.VMEM((CHUNK * LANES,), jnp.int32),
            pltpu.VMEM((CHUNK, PAD_LEN), jnp.int32),
            pltpu.VMEM((CHUNK, PAD_LEN), jnp.int32),
            pltpu.SemaphoreType.DMA,
        ],
    )
    def _history_view(hist_hbm, anchor_hbm, tgtb_hbm, padded_hbm, mask_hbm,
                      idx_v, tgt_v, rows_v, mask_v, sem):
        wid = lax.axis_index("s") * NUM_CORES + lax.axis_index("c")

        def row_body(r, _):
            tgt = tgt_v[pl.ds(r * LANES, LANES)]
            for off in _WIN_OFFS:
                h = rows_v[r, pl.ds(off, LANES)]
                keep = ~((h == tgt) | (h == PADDING_IDX))
                rows_v[r, pl.ds(off, LANES)] = jnp.where(keep, h, PADDING_IDX)
                mask_v[r, pl.ds(off, LANES)] = keep.astype(jnp.int32)
            return 0

        for c in range(NCHUNK):
            base = wid * ROWS_PER_W + c * CHUNK
            pltpu.sync_copy(anchor_hbm.at[pl.ds(base, CHUNK)], idx_v)
            pltpu.sync_copy(tgtb_hbm.at[pl.ds(base * LANES, CHUNK * LANES)],
                            tgt_v)
            pltpu.async_copy(hist_hbm.at[idx_v], rows_v, sem).wait()
            lax.fori_loop(0, CHUNK, row_body, 0)
            pltpu.sync_copy(rows_v, padded_hbm.at[pl.ds(base, CHUNK)])
            pltpu.sync_copy(mask_v, mask_hbm.at[pl.ds(base, CHUNK)])

    return _history_view


def kernel(histories, anchor_idx, target_idx):
    out_dtype = histories.dtype
    hist_t = histories.astype(jnp.int32).T
    hist_pad = _build_transpose()(hist_t)
    tgt_flat = jnp.broadcast_to(
        target_idx.astype(jnp.int32)[:, None],
        (BATCH, LANES)).reshape(BATCH * LANES)
    padded, mask_i32 = _build_history_view()(
        hist_pad,
        anchor_idx.astype(jnp.int32),
        tgt_flat,
    )
    return (padded[:, :HIST_LEN].astype(out_dtype),
            mask_i32[:, :HIST_LEN].astype(jnp.bool_))


# speedup vs baseline: 5.5497x; 1.2237x over previous
import functools

import jax
import jax.numpy as jnp
from jax import lax
from jax.experimental import pallas as pl
from jax.experimental.pallas import tpu as pltpu
from jax.experimental.pallas import tpu_sc as plsc

VOCAB = 100000
HIST_LEN = 200
PAD_LEN = 256
BATCH = 16384
PADDING_IDX = 0

NUM_CORES = 2
NUM_SUBCORES = 16
NW = NUM_CORES * NUM_SUBCORES
ROWS_PER_W = BATCH // NW
CHUNK = 128
NCHUNK = ROWS_PER_W // CHUNK

TBLK = 2048
PBLK = 2048


def _transpose_body(in_ref, out_ref):
    x = in_ref[...]
    out_ref[:, :HIST_LEN] = x.T


@functools.cache
def _build_transpose():
    grid = (VOCAB + TBLK - 1) // TBLK
    return pl.pallas_call(
        _transpose_body,
        grid=(grid,),
        in_specs=[pl.BlockSpec((HIST_LEN, TBLK), lambda i: (0, i))],
        out_specs=pl.BlockSpec((TBLK, PAD_LEN), lambda i: (i, 0)),
        out_shape=jax.ShapeDtypeStruct((VOCAB, PAD_LEN), jnp.int32),
    )


@functools.cache
def _build_gather():
    mesh = plsc.VectorSubcoreMesh(core_axis_name="c", subcore_axis_name="s")

    @functools.partial(
        pl.kernel,
        out_type=jax.ShapeDtypeStruct((BATCH, PAD_LEN), jnp.int32),
        mesh=mesh,
        compiler_params=pltpu.CompilerParams(
            use_tc_tiling_on_sc=True,
            needs_layout_passes=False,
        ),
        scratch_types=[
            pltpu.VMEM((NCHUNK, CHUNK), jnp.int32),
            pltpu.VMEM((CHUNK, PAD_LEN), jnp.int32),
            pltpu.VMEM((CHUNK, PAD_LEN), jnp.int32),
            pltpu.SemaphoreType.DMA,
            pltpu.SemaphoreType.DMA,
            pltpu.SemaphoreType.DMA,
            pltpu.SemaphoreType.DMA,
        ],
    )
    def _gather(hist_hbm, anchor_hbm, out_hbm,
                idx_v, rows0, rows1, g0, g1, w0, w1):
        wid = lax.axis_index("s") * NUM_CORES + lax.axis_index("c")
        base = wid * ROWS_PER_W
        for c in range(NCHUNK):
            pltpu.sync_copy(anchor_hbm.at[pl.ds(base + c * CHUNK, CHUNK)],
                            idx_v.at[c])
        rows = (rows0, rows1)
        gsem = (g0, g1)
        wsem = (w0, w1)
        ghandles = [None] * NCHUNK
        whandles = [None] * NCHUNK
        ghandles[0] = pltpu.async_copy(hist_hbm.at[idx_v.at[0]], rows[0],
                                       gsem[0])
        for c in range(NCHUNK):
            cur = c & 1
            ghandles[c].wait()
            whandles[c] = pltpu.async_copy(
                rows[cur], out_hbm.at[pl.ds(base + c * CHUNK, CHUNK)],
                wsem[cur])
            if c + 1 < NCHUNK:
                if c >= 1:
                    whandles[c - 1].wait()
                ghandles[c + 1] = pltpu.async_copy(
                    hist_hbm.at[idx_v.at[c + 1]], rows[1 - cur],
                    gsem[1 - cur])
        whandles[NCHUNK - 2].wait()
        whandles[NCHUNK - 1].wait()

    return _gather


def _mask_body(g_ref, t_ref, p_ref, m_ref):
    ht = g_ref[:, :HIST_LEN].T
    tt = t_ref[...]
    keep = (ht != tt) & (ht != PADDING_IDX)
    p_ref[...] = jnp.where(keep, ht, PADDING_IDX)
    m_ref[...] = keep


@functools.cache
def _build_mask():
    grid = BATCH // PBLK
    return pl.pallas_call(
        _mask_body,
        grid=(grid,),
        in_specs=[
            pl.BlockSpec((PBLK, PAD_LEN), lambda i: (i, 0)),
            pl.BlockSpec((1, PBLK), lambda i: (0, i)),
        ],
        out_specs=[
            pl.BlockSpec((HIST_LEN, PBLK), lambda i: (0, i)),
            pl.BlockSpec((HIST_LEN, PBLK), lambda i: (0, i)),
        ],
        out_shape=(
            jax.ShapeDtypeStruct((HIST_LEN, BATCH), jnp.int32),
            jax.ShapeDtypeStruct((HIST_LEN, BATCH), jnp.bool_),
        ),
    )


def kernel(histories, anchor_idx, target_idx):
    out_dtype = histories.dtype
    hist_t = histories.astype(jnp.int32).T
    hist_pad = _build_transpose()(hist_t)
    gathered = _build_gather()(hist_pad, anchor_idx.astype(jnp.int32))
    tgt_row = target_idx.astype(jnp.int32).reshape(1, BATCH)
    padded_t, mask_t = _build_mask()(gathered, tgt_row)
    return padded_t.T.astype(out_dtype), mask_t.T


# speedup vs baseline: 6.2233x; 1.1214x over previous
import functools

import jax
import jax.numpy as jnp
from jax import lax
from jax.experimental import pallas as pl
from jax.experimental.pallas import tpu as pltpu
from jax.experimental.pallas import tpu_sc as plsc

VOCAB = 100000
HIST_LEN = 200
PAD_LEN = 256
BATCH = 16384
PADDING_IDX = 0

NUM_CORES = 2
NUM_SUBCORES = 16
NW = NUM_CORES * NUM_SUBCORES
ROWS_PER_W = BATCH // NW
CHUNK = 128
NCHUNK = ROWS_PER_W // CHUNK

TBLK = 4096
PBLK = 4096


def _transpose_body(in_ref, out_ref):
    x = in_ref[...]
    out_ref[:, :HIST_LEN] = x.T


@functools.cache
def _build_transpose():
    grid = (VOCAB + TBLK - 1) // TBLK
    return pl.pallas_call(
        _transpose_body,
        grid=(grid,),
        in_specs=[pl.BlockSpec((HIST_LEN, TBLK), lambda i: (0, i))],
        out_specs=pl.BlockSpec((TBLK, PAD_LEN), lambda i: (i, 0)),
        out_shape=jax.ShapeDtypeStruct((VOCAB, PAD_LEN), jnp.int32),
    )


@functools.cache
def _build_gather():
    mesh = plsc.VectorSubcoreMesh(core_axis_name="c", subcore_axis_name="s")

    @functools.partial(
        pl.kernel,
        out_type=jax.ShapeDtypeStruct((BATCH, PAD_LEN), jnp.int32),
        mesh=mesh,
        compiler_params=pltpu.CompilerParams(
            use_tc_tiling_on_sc=True,
            needs_layout_passes=False,
        ),
        scratch_types=[
            pltpu.VMEM((NCHUNK, CHUNK), jnp.int32),
            pltpu.VMEM((CHUNK, PAD_LEN), jnp.int32),
            pltpu.VMEM((CHUNK, PAD_LEN), jnp.int32),
            pltpu.SemaphoreType.DMA,
            pltpu.SemaphoreType.DMA,
            pltpu.SemaphoreType.DMA,
            pltpu.SemaphoreType.DMA,
        ],
    )
    def _gather(hist_hbm, anchor_hbm, out_hbm,
                idx_v, rows0, rows1, g0, g1, w0, w1):
        wid = lax.axis_index("s") * NUM_CORES + lax.axis_index("c")
        base = wid * ROWS_PER_W
        for c in range(NCHUNK):
            pltpu.sync_copy(anchor_hbm.at[pl.ds(base + c * CHUNK, CHUNK)],
                            idx_v.at[c])
        rows = (rows0, rows1)
        gsem = (g0, g1)
        wsem = (w0, w1)
        ghandles = [None] * NCHUNK
        whandles = [None] * NCHUNK
        ghandles[0] = pltpu.async_copy(hist_hbm.at[idx_v.at[0]], rows[0],
                                       gsem[0])
        for c in range(NCHUNK):
            cur = c & 1
            ghandles[c].wait()
            whandles[c] = pltpu.async_copy(
                rows[cur], out_hbm.at[pl.ds(base + c * CHUNK, CHUNK)],
                wsem[cur])
            if c + 1 < NCHUNK:
                if c >= 1:
                    whandles[c - 1].wait()
                ghandles[c + 1] = pltpu.async_copy(
                    hist_hbm.at[idx_v.at[c + 1]], rows[1 - cur],
                    gsem[1 - cur])
        whandles[NCHUNK - 2].wait()
        whandles[NCHUNK - 1].wait()

    return _gather


def _mask_body(g_ref, t_ref, p_ref, m_ref):
    ht = g_ref[:, :HIST_LEN].T
    tt = t_ref[...]
    keep = (ht != tt) & (ht != PADDING_IDX)
    p_ref[...] = jnp.where(keep, ht, PADDING_IDX)
    m_ref[...] = keep


@functools.cache
def _build_mask():
    grid = BATCH // PBLK
    return pl.pallas_call(
        _mask_body,
        grid=(grid,),
        in_specs=[
            pl.BlockSpec((PBLK, PAD_LEN), lambda i: (i, 0)),
            pl.BlockSpec((1, PBLK), lambda i: (0, i)),
        ],
        out_specs=[
            pl.BlockSpec((HIST_LEN, PBLK), lambda i: (0, i)),
            pl.BlockSpec((HIST_LEN, PBLK), lambda i: (0, i)),
        ],
        out_shape=(
            jax.ShapeDtypeStruct((HIST_LEN, BATCH), jnp.int32),
            jax.ShapeDtypeStruct((HIST_LEN, BATCH), jnp.bool_),
        ),
    )


def kernel(histories, anchor_idx, target_idx):
    out_dtype = histories.dtype
    hist_t = histories.astype(jnp.int32).T
    hist_pad = _build_transpose()(hist_t)
    gathered = _build_gather()(hist_pad, anchor_idx.astype(jnp.int32))
    tgt_row = target_idx.astype(jnp.int32).reshape(1, BATCH)
    padded_t, mask_t = _build_mask()(gathered, tgt_row)
    return padded_t.T.astype(out_dtype), mask_t.T


# speedup vs baseline: 6.3385x; 1.0185x over previous
import functools

import jax
import jax.numpy as jnp
from jax import lax
from jax.experimental import pallas as pl
from jax.experimental.pallas import tpu as pltpu
from jax.experimental.pallas import tpu_sc as plsc

VOCAB = 100000
HIST_LEN = 200
PAD_LEN = 256
BATCH = 16384
PADDING_IDX = 0

NUM_CORES = 2
NUM_SUBCORES = 16
NW = NUM_CORES * NUM_SUBCORES
ROWS_PER_W = BATCH // NW
CHUNK = 128
NCHUNK = ROWS_PER_W // CHUNK

TBLK = 8192
PBLK = 4096


def _transpose_body(in_ref, out_ref):
    x = in_ref[...]
    out_ref[:, :HIST_LEN] = x.T


@functools.cache
def _build_transpose():
    grid = (VOCAB + TBLK - 1) // TBLK
    return pl.pallas_call(
        _transpose_body,
        grid=(grid,),
        in_specs=[pl.BlockSpec((HIST_LEN, TBLK), lambda i: (0, i))],
        out_specs=pl.BlockSpec((TBLK, PAD_LEN), lambda i: (i, 0)),
        out_shape=jax.ShapeDtypeStruct((VOCAB, PAD_LEN), jnp.int32),
    )


@functools.cache
def _build_gather():
    mesh = plsc.VectorSubcoreMesh(core_axis_name="c", subcore_axis_name="s")

    @functools.partial(
        pl.kernel,
        out_type=jax.ShapeDtypeStruct((BATCH, PAD_LEN), jnp.int32),
        mesh=mesh,
        compiler_params=pltpu.CompilerParams(
            use_tc_tiling_on_sc=True,
            needs_layout_passes=False,
        ),
        scratch_types=[
            pltpu.VMEM((NCHUNK, CHUNK), jnp.int32),
            pltpu.VMEM((CHUNK, PAD_LEN), jnp.int32),
            pltpu.VMEM((CHUNK, PAD_LEN), jnp.int32),
            pltpu.SemaphoreType.DMA,
            pltpu.SemaphoreType.DMA,
            pltpu.SemaphoreType.DMA,
            pltpu.SemaphoreType.DMA,
        ],
    )
    def _gather(hist_hbm, anchor_hbm, out_hbm,
                idx_v, rows0, rows1, g0, g1, w0, w1):
        wid = lax.axis_index("s") * NUM_CORES + lax.axis_index("c")
        base = wid * ROWS_PER_W
        for c in range(NCHUNK):
            pltpu.sync_copy(anchor_hbm.at[pl.ds(base + c * CHUNK, CHUNK)],
                            idx_v.at[c])
        rows = (rows0, rows1)
        gsem = (g0, g1)
        wsem = (w0, w1)
        ghandles = [None] * NCHUNK
        whandles = [None] * NCHUNK
        ghandles[0] = pltpu.async_copy(hist_hbm.at[idx_v.at[0]], rows[0],
                                       gsem[0])
        for c in range(NCHUNK):
            cur = c & 1
            ghandles[c].wait()
            whandles[c] = pltpu.async_copy(
                rows[cur], out_hbm.at[pl.ds(base + c * CHUNK, CHUNK)],
                wsem[cur])
            if c + 1 < NCHUNK:
                if c >= 1:
                    whandles[c - 1].wait()
                ghandles[c + 1] = pltpu.async_copy(
                    hist_hbm.at[idx_v.at[c + 1]], rows[1 - cur],
                    gsem[1 - cur])
        whandles[NCHUNK - 2].wait()
        whandles[NCHUNK - 1].wait()

    return _gather


def _mask_body(g_ref, t_ref, p_ref, m_ref):
    ht = g_ref[:, :HIST_LEN].T
    tt = t_ref[...]
    keep = (ht != tt) & (ht != PADDING_IDX)
    p_ref[...] = jnp.where(keep, ht, PADDING_IDX)
    m_ref[...] = keep


@functools.cache
def _build_mask():
    grid = BATCH // PBLK
    return pl.pallas_call(
        _mask_body,
        grid=(grid,),
        in_specs=[
            pl.BlockSpec((PBLK, PAD_LEN), lambda i: (i, 0)),
            pl.BlockSpec((1, PBLK), lambda i: (0, i)),
        ],
        out_specs=[
            pl.BlockSpec((HIST_LEN, PBLK), lambda i: (0, i)),
            pl.BlockSpec((HIST_LEN, PBLK), lambda i: (0, i)),
        ],
        out_shape=(
            jax.ShapeDtypeStruct((HIST_LEN, BATCH), jnp.int32),
            jax.ShapeDtypeStruct((HIST_LEN, BATCH), jnp.bool_),
        ),
    )


def kernel(histories, anchor_idx, target_idx):
    out_dtype = histories.dtype
    hist_t = histories.astype(jnp.int32).T
    hist_pad = _build_transpose()(hist_t)
    gathered = _build_gather()(hist_pad, anchor_idx.astype(jnp.int32))
    tgt_row = target_idx.astype(jnp.int32).reshape(1, BATCH)
    padded_t, mask_t = _build_mask()(gathered, tgt_row)
    return padded_t.T.astype(out_dtype), mask_t.T


# speedup vs baseline: 6.4225x; 1.0132x over previous
import functools

import jax
import jax.numpy as jnp
from jax import lax
from jax.experimental import pallas as pl
from jax.experimental.pallas import tpu as pltpu
from jax.experimental.pallas import tpu_sc as plsc

VOCAB = 100000
HIST_LEN = 200
PAD_LEN = 256
BATCH = 16384
PADDING_IDX = 0

NUM_CORES = 2
NUM_SUBCORES = 16
NW = NUM_CORES * NUM_SUBCORES
ROWS_PER_W = BATCH // NW
CHUNK = 128
NCHUNK = ROWS_PER_W // CHUNK

TBLK = 12288
PBLK = 8192


def _transpose_body(in_ref, out_ref):
    x = in_ref[...]
    out_ref[:, :HIST_LEN] = x.T


@functools.cache
def _build_transpose():
    grid = (VOCAB + TBLK - 1) // TBLK
    return pl.pallas_call(
        _transpose_body,
        grid=(grid,),
        in_specs=[pl.BlockSpec((HIST_LEN, TBLK), lambda i: (0, i))],
        out_specs=pl.BlockSpec((TBLK, PAD_LEN), lambda i: (i, 0)),
        out_shape=jax.ShapeDtypeStruct((VOCAB, PAD_LEN), jnp.int32),
    )


@functools.cache
def _build_gather():
    mesh = plsc.VectorSubcoreMesh(core_axis_name="c", subcore_axis_name="s")

    @functools.partial(
        pl.kernel,
        out_type=jax.ShapeDtypeStruct((BATCH, PAD_LEN), jnp.int32),
        mesh=mesh,
        compiler_params=pltpu.CompilerParams(
            use_tc_tiling_on_sc=True,
            needs_layout_passes=False,
        ),
        scratch_types=[
            pltpu.VMEM((NCHUNK, CHUNK), jnp.int32),
            pltpu.VMEM((CHUNK, PAD_LEN), jnp.int32),
            pltpu.VMEM((CHUNK, PAD_LEN), jnp.int32),
            pltpu.SemaphoreType.DMA,
            pltpu.SemaphoreType.DMA,
            pltpu.SemaphoreType.DMA,
            pltpu.SemaphoreType.DMA,
        ],
    )
    def _gather(hist_hbm, anchor_hbm, out_hbm,
                idx_v, rows0, rows1, g0, g1, w0, w1):
        wid = lax.axis_index("s") * NUM_CORES + lax.axis_index("c")
        base = wid * ROWS_PER_W
        for c in range(NCHUNK):
            pltpu.sync_copy(anchor_hbm.at[pl.ds(base + c * CHUNK, CHUNK)],
                            idx_v.at[c])
        rows = (rows0, rows1)
        gsem = (g0, g1)
        wsem = (w0, w1)
        ghandles = [None] * NCHUNK
        whandles = [None] * NCHUNK
        ghandles[0] = pltpu.async_copy(hist_hbm.at[idx_v.at[0]], rows[0],
                                       gsem[0])
        for c in range(NCHUNK):
            cur = c & 1
            ghandles[c].wait()
            whandles[c] = pltpu.async_copy(
                rows[cur], out_hbm.at[pl.ds(base + c * CHUNK, CHUNK)],
                wsem[cur])
            if c + 1 < NCHUNK:
                if c >= 1:
                    whandles[c - 1].wait()
                ghandles[c + 1] = pltpu.async_copy(
                    hist_hbm.at[idx_v.at[c + 1]], rows[1 - cur],
                    gsem[1 - cur])
        whandles[NCHUNK - 2].wait()
        whandles[NCHUNK - 1].wait()

    return _gather


def _mask_body(g_ref, t_ref, p_ref, m_ref):
    ht = g_ref[:, :HIST_LEN].T
    tt = t_ref[...]
    keep = (ht != tt) & (ht != PADDING_IDX)
    p_ref[...] = jnp.where(keep, ht, PADDING_IDX)
    m_ref[...] = keep


@functools.cache
def _build_mask():
    grid = BATCH // PBLK
    return pl.pallas_call(
        _mask_body,
        grid=(grid,),
        in_specs=[
            pl.BlockSpec((PBLK, PAD_LEN), lambda i: (i, 0)),
            pl.BlockSpec((1, PBLK), lambda i: (0, i)),
        ],
        out_specs=[
            pl.BlockSpec((HIST_LEN, PBLK), lambda i: (0, i)),
            pl.BlockSpec((HIST_LEN, PBLK), lambda i: (0, i)),
        ],
        out_shape=(
            jax.ShapeDtypeStruct((HIST_LEN, BATCH), jnp.int32),
            jax.ShapeDtypeStruct((HIST_LEN, BATCH), jnp.bool_),
        ),
    )


def kernel(histories, anchor_idx, target_idx):
    out_dtype = histories.dtype
    hist_t = histories.astype(jnp.int32).T
    hist_pad = _build_transpose()(hist_t)
    gathered = _build_gather()(hist_pad, anchor_idx.astype(jnp.int32))
    tgt_row = target_idx.astype(jnp.int32).reshape(1, BATCH)
    padded_t, mask_t = _build_mask()(gathered, tgt_row)
    return padded_t.T.astype(out_dtype), mask_t.T
